# baseline (speedup 1.0000x reference)
import jax
import jax.numpy as jnp
from jax import lax
from jax.experimental import pallas as pl
from jax.experimental.pallas import tpu as pltpu
from jax.experimental.pallas import tpu_sc as plsc

NUM_SEG = 512
NUM_NODES = 100000
FEAT = 128
CHUNK = 80
GRP = 4
GROWS = GRP * CHUNK
NCHUNKS = NUM_NODES // CHUNK
NW = 32
BASE_PER_W = NCHUNKS // NW
EXTRA = NCHUNKS - BASE_PER_W * NW
NGROUP = (BASE_PER_W + 1 + GRP - 1) // GRP
SEG_PER_TILE = NUM_SEG // 16


def _seg_body(attr_hbm, batch_hbm, psum_hbm, pcnt_hbm,
              chunks, idss, ones_v, stage_v, cstage_v,
              acc_sh, cnt_sh, sgs, sss):
    cid = lax.axis_index("c")
    sid = lax.axis_index("s")
    w = cid * 16 + sid

    zeros16 = jnp.zeros((16,), jnp.float32)
    ones16 = jnp.ones((16,), jnp.float32)
    for i in range(SEG_PER_TILE):
        for j in range(FEAT // 16):
            stage_v[i, pl.ds(j * 16, 16)] = zeros16
    for i in range(SEG_PER_TILE // 16):
        cstage_v[pl.ds(i * 16, 16)] = zeros16
    for i in range(CHUNK // 16):
        ones_v[pl.ds(i * 16, 16)] = ones16

    pltpu.sync_copy(stage_v, acc_sh.at[pl.ds(sid * SEG_PER_TILE, SEG_PER_TILE)])
    pltpu.sync_copy(cstage_v, cnt_sh.at[pl.ds(sid * SEG_PER_TILE, SEG_PER_TILE)])
    plsc.subcore_barrier()

    full = w < EXTRA
    start_w = BASE_PER_W * w + jnp.minimum(w, EXTRA)

    def gather(g, b):
        base = (start_w + g * GRP) * CHUNK
        if g < NGROUP - 1:
            pltpu.async_copy(attr_hbm.at[pl.ds(base, GROWS)], chunks[b], sgs[b])
        else:
            @pl.when(full)
            def _():
                pltpu.async_copy(attr_hbm.at[pl.ds(base, GROWS)], chunks[b], sgs[b])

            @pl.when(jnp.logical_not(full))
            def _():
                pltpu.async_copy(attr_hbm.at[pl.ds(base, GROWS - CHUNK)],
                                 chunks[b].at[pl.ds(0, GROWS - CHUNK)], sgs[b])
        for k in range(GRP):
            if g * GRP + k <= BASE_PER_W - 1:
                pltpu.async_copy(batch_hbm.at[pl.ds(base + k * CHUNK, CHUNK)],
                                 idss[b][k], sgs[b])
            else:
                @pl.when(full)
                def _():
                    pltpu.async_copy(batch_hbm.at[pl.ds(base + k * CHUNK, CHUNK)],
                                     idss[b][k], sgs[b])

    def gather_wait(g, b):
        if g < NGROUP - 1:
            pltpu.make_async_copy(attr_hbm.at[pl.ds(0, GROWS)], chunks[b],
                                  sgs[b]).wait()
        else:
            @pl.when(full)
            def _():
                pltpu.make_async_copy(attr_hbm.at[pl.ds(0, GROWS)], chunks[b],
                                      sgs[b]).wait()

            @pl.when(jnp.logical_not(full))
            def _():
                pltpu.make_async_copy(attr_hbm.at[pl.ds(0, GROWS - CHUNK)],
                                      chunks[b].at[pl.ds(0, GROWS - CHUNK)],
                                      sgs[b]).wait()
        for k in range(GRP):
            if g * GRP + k <= BASE_PER_W - 1:
                pltpu.make_async_copy(batch_hbm.at[pl.ds(0, CHUNK)], idss[b][k],
                                      sgs[b]).wait()
            else:
                @pl.when(full)
                def _():
                    pltpu.make_async_copy(batch_hbm.at[pl.ds(0, CHUNK)], idss[b][k],
                                          sgs[b]).wait()

    def scatter(g, b):
        for k in range(GRP):
            src = chunks[b].at[pl.ds(k * CHUNK, CHUNK)]
            if g * GRP + k <= BASE_PER_W - 1:
                pltpu.async_copy(src, acc_sh.at[idss[b][k]], sss[b], add=True)
                pltpu.async_copy(ones_v, cnt_sh.at[idss[b][k]], sss[b], add=True)
            else:
                @pl.when(full)
                def _():
                    pltpu.async_copy(src, acc_sh.at[idss[b][k]], sss[b], add=True)
                    pltpu.async_copy(ones_v, cnt_sh.at[idss[b][k]], sss[b], add=True)

    def scatter_wait(g, b):
        for k in range(GRP):
            src = chunks[b].at[pl.ds(k * CHUNK, CHUNK)]
            if g * GRP + k <= BASE_PER_W - 1:
                pltpu.make_async_copy(src, acc_sh.at[pl.ds(0, CHUNK)], sss[b]).wait()
                pltpu.make_async_copy(ones_v, cnt_sh.at[pl.ds(0, CHUNK)],
                                      sss[b]).wait()
            else:
                @pl.when(full)
                def _():
                    pltpu.make_async_copy(src, acc_sh.at[pl.ds(0, CHUNK)],
                                          sss[b]).wait()
                    pltpu.make_async_copy(ones_v, cnt_sh.at[pl.ds(0, CHUNK)],
                                          sss[b]).wait()

    gather(0, 0)
    gather(1, 1)
    for g in range(NGROUP):
        b = g % 2
        gather_wait(g, b)
        scatter(g, b)
        if g + 2 < NGROUP:
            scatter_wait(g, b)
            gather(g + 2, b)
    scatter_wait(NGROUP - 2, (NGROUP - 2) % 2)
    scatter_wait(NGROUP - 1, (NGROUP - 1) % 2)

    plsc.subcore_barrier()

    row = sid * SEG_PER_TILE
    pltpu.sync_copy(acc_sh.at[pl.ds(row, SEG_PER_TILE)], stage_v)
    pltpu.sync_copy(stage_v, psum_hbm.at[pl.ds(cid * NUM_SEG + row, SEG_PER_TILE)])
    pltpu.sync_copy(cnt_sh.at[pl.ds(row, SEG_PER_TILE)], cstage_v)
    pltpu.sync_copy(cstage_v, pcnt_hbm.at[cid, pl.ds(row, SEG_PER_TILE)])


def _body_wrapper(attr_hbm, batch_hbm, psum_hbm, pcnt_hbm,
                  c0, c1, i00, i01, i02, i03, i10, i11, i12, i13,
                  ones_v, stage_v, cstage_v, acc_sh, cnt_sh, g0, g1, s0, s1):
    _seg_body(attr_hbm, batch_hbm, psum_hbm, pcnt_hbm,
              [c0, c1], [[i00, i01, i02, i03], [i10, i11, i12, i13]],
              ones_v, stage_v, cstage_v, acc_sh, cnt_sh, [g0, g1], [s0, s1])


_seg_kernel = pl.kernel(
    _body_wrapper,
    out_type=[
        jax.ShapeDtypeStruct((2 * NUM_SEG, FEAT), jnp.float32),
        jax.ShapeDtypeStruct((16, NUM_SEG), jnp.float32),
    ],
    mesh=plsc.VectorSubcoreMesh(core_axis_name="c", subcore_axis_name="s"),
    scratch_types=(
        [pltpu.VMEM((GROWS, FEAT), jnp.float32)] * 2
        + [pltpu.VMEM((CHUNK,), jnp.int32)] * 8
        + [
            pltpu.VMEM((CHUNK,), jnp.float32),
            pltpu.VMEM((SEG_PER_TILE, FEAT), jnp.float32),
            pltpu.VMEM((SEG_PER_TILE,), jnp.float32),
            pltpu.VMEM_SHARED((NUM_SEG, FEAT), jnp.float32),
            pltpu.VMEM_SHARED((NUM_SEG,), jnp.float32),
        ]
        + [pltpu.SemaphoreType.DMA] * 4
    ),
)


def _combine_body(ps_ref, pc_ref, o_ref):
    s = ps_ref[0:NUM_SEG, :] + ps_ref[NUM_SEG:2 * NUM_SEG, :]
    ct = jnp.transpose(pc_ref[...], (1, 0))
    c = ct[:, 0:1] + ct[:, 1:2]
    o_ref[...] = s / jnp.maximum(c, 1.0)


def kernel(node_attr, batch):
    psum, pcnt = _seg_kernel(node_attr, batch)
    mean = pl.pallas_call(
        _combine_body,
        out_shape=jax.ShapeDtypeStruct((NUM_SEG, FEAT), jnp.float32),
    )(psum, pcnt)
    return mean.reshape(-1)


# speedup vs baseline: 1.1852x; 1.1852x over previous
import jax
import jax.numpy as jnp
from jax import lax
from jax.experimental import pallas as pl
from jax.experimental.pallas import tpu as pltpu
from jax.experimental.pallas import tpu_sc as plsc

NUM_SEG = 512
NUM_NODES = 100000
FEAT = 128
CHUNK = 80
NCHUNKS = NUM_NODES // CHUNK
NW = 32
PER_W = 40
LAST_W = NCHUNKS - PER_W * (NW - 1)
NBUF = 4
RSTEPS = PER_W // NBUF
SEG_PER_TILE = NUM_SEG // 16


def _seg_body(attr_hbm, batch_hbm, psum_hbm, pcnt_hbm,
              chunks, ids2d, ones_v, stage_v, cstage_v,
              acc_sh, cnt_sh, sgs, sss):
    cid = lax.axis_index("c")
    sid = lax.axis_index("s")
    w = cid * 16 + sid

    zeros16 = jnp.zeros((16,), jnp.float32)
    ones16 = jnp.ones((16,), jnp.float32)
    for i in range(SEG_PER_TILE):
        for j in range(FEAT // 16):
            stage_v[i, pl.ds(j * 16, 16)] = zeros16
    for i in range(SEG_PER_TILE // 16):
        cstage_v[pl.ds(i * 16, 16)] = zeros16
    for i in range(CHUNK // 16):
        ones_v[pl.ds(i * 16, 16)] = ones16

    pltpu.sync_copy(stage_v, acc_sh.at[pl.ds(sid * SEG_PER_TILE, SEG_PER_TILE)])
    pltpu.sync_copy(cstage_v, cnt_sh.at[pl.ds(sid * SEG_PER_TILE, SEG_PER_TILE)])
    plsc.subcore_barrier()

    last = w == NW - 1
    n_w = jnp.where(last, LAST_W, PER_W)
    start_w = PER_W * w

    pltpu.sync_copy(batch_hbm.at[pl.ds(start_w, PER_W)], ids2d)

    def gather(i, b):
        base = (start_w + i) * CHUNK
        pltpu.async_copy(attr_hbm.at[pl.ds(base, CHUNK)], chunks[b], sgs[b])

    def gather_wait(b):
        pltpu.make_async_copy(attr_hbm.at[pl.ds(0, CHUNK)], chunks[b], sgs[b]).wait()

    def scatter(i, b):
        pltpu.async_copy(chunks[b], acc_sh.at[ids2d.at[i]], sss[b], add=True)
        pltpu.async_copy(ones_v, cnt_sh.at[ids2d.at[i]], sss[b], add=True)

    def scatter_wait(b):
        pltpu.make_async_copy(chunks[b], acc_sh.at[pl.ds(0, CHUNK)], sss[b]).wait()
        pltpu.make_async_copy(ones_v, cnt_sh.at[pl.ds(0, CHUNK)], sss[b]).wait()

    for b in range(NBUF):
        gather(b, b)

    for t in range(RSTEPS):
        for b in range(NBUF):
            i = NBUF * t + b

            @pl.when(i < n_w)
            def _():
                gather_wait(b)
                scatter(i, b)

        for b in range(NBUF):
            i = NBUF * t + b

            @pl.when(i + NBUF < n_w)
            def _():
                scatter_wait(b)
                gather(i + NBUF, b)

    for b in range(NBUF):
        scatter_wait(b)

    plsc.subcore_barrier()

    row = sid * SEG_PER_TILE
    pltpu.sync_copy(acc_sh.at[pl.ds(row, SEG_PER_TILE)], stage_v)
    pltpu.sync_copy(stage_v, psum_hbm.at[pl.ds(cid * NUM_SEG + row, SEG_PER_TILE)])
    pltpu.sync_copy(cnt_sh.at[pl.ds(row, SEG_PER_TILE)], cstage_v)
    pltpu.sync_copy(cstage_v, pcnt_hbm.at[cid, pl.ds(row, SEG_PER_TILE)])


def _body_wrapper(attr_hbm, batch_hbm, psum_hbm, pcnt_hbm,
                  c0, c1, c2, c3, ids2d, ones_v, stage_v, cstage_v,
                  acc_sh, cnt_sh, g0, g1, g2, g3, s0, s1, s2, s3):
    _seg_body(attr_hbm, batch_hbm, psum_hbm, pcnt_hbm,
              [c0, c1, c2, c3], ids2d, ones_v, stage_v, cstage_v,
              acc_sh, cnt_sh, [g0, g1, g2, g3], [s0, s1, s2, s3])


_seg_kernel = pl.kernel(
    _body_wrapper,
    out_type=[
        jax.ShapeDtypeStruct((2 * NUM_SEG, FEAT), jnp.float32),
        jax.ShapeDtypeStruct((16, NUM_SEG), jnp.float32),
    ],
    mesh=plsc.VectorSubcoreMesh(core_axis_name="c", subcore_axis_name="s"),
    scratch_types=(
        [pltpu.VMEM((CHUNK, FEAT), jnp.float32)] * NBUF
        + [pltpu.VMEM((PER_W, CHUNK), jnp.int32)]
        + [
            pltpu.VMEM((CHUNK,), jnp.float32),
            pltpu.VMEM((SEG_PER_TILE, FEAT), jnp.float32),
            pltpu.VMEM((SEG_PER_TILE,), jnp.float32),
            pltpu.VMEM_SHARED((NUM_SEG, FEAT), jnp.float32),
            pltpu.VMEM_SHARED((NUM_SEG,), jnp.float32),
        ]
        + [pltpu.SemaphoreType.DMA] * (2 * NBUF)
    ),
)


def _combine_body(ps_ref, pc_ref, o_ref):
    s = ps_ref[0:NUM_SEG, :] + ps_ref[NUM_SEG:2 * NUM_SEG, :]
    ct = jnp.transpose(pc_ref[...], (1, 0))
    c = ct[:, 0:1] + ct[:, 1:2]
    o_ref[...] = s / jnp.maximum(c, 1.0)


def kernel(node_attr, batch):
    batch2d = jnp.pad(batch.reshape(NCHUNKS, CHUNK),
                      ((0, NW * PER_W - NCHUNKS), (0, 0)))
    psum, pcnt = _seg_kernel(node_attr, batch2d)
    mean = pl.pallas_call(
        _combine_body,
        out_shape=jax.ShapeDtypeStruct((NUM_SEG, FEAT), jnp.float32),
    )(psum, pcnt)
    return mean.reshape(-1)
